# baseline (speedup 1.0000x reference)
import functools

import jax
import jax.numpy as jnp
from jax import lax
from jax.experimental import pallas as pl
from jax.experimental.pallas import tpu as pltpu
from jax.experimental.pallas import tpu_sc as plsc

VOCAB = 100001
EMBED_DIM = 32
BATCH = 16384

_info = plsc.get_sparse_core_info()
_NC = _info.num_cores
_NS = _info.num_subcores
_NW = _NC * _NS
_N = BATCH // _NW

_COLS = 100096
_UNITS = _COLS // 128
_FULL_UNITS = VOCAB // 128
_TAIL = VOCAB - _FULL_UNITS * 128
_UPW = (_UNITS + _NW - 1) // _NW
_RP = 33

_R = EMBED_DIM // 8
_C = BATCH // 128
_CW = _C // _NW
_XP = 129

_mesh = plsc.VectorSubcoreMesh(core_axis_name="c", subcore_axis_name="s")


@functools.partial(
    pl.kernel,
    mesh=_mesh,
    out_type=jax.ShapeDtypeStruct((_R, _C, 8, 128), jnp.float32),
    scratch_types=[
        pltpu.MemorySpace.HBM((_COLS, _RP), jnp.float32),
        pltpu.VMEM((EMBED_DIM, 128), jnp.float32),
        pltpu.VMEM((128, _RP), jnp.float32),
        pltpu.VMEM((40, 128), jnp.float32),
        pltpu.VMEM((_N,), jnp.int32),
        pltpu.VMEM((_N, _RP), jnp.float32),
        pltpu.VMEM((_R, _CW, 8, _XP), jnp.float32),
        pltpu.SemaphoreType.DMA,
        pltpu.SemaphoreType.REGULAR,
    ],
    compiler_params=pltpu.CompilerParams(
        use_tc_tiling_on_sc=True, needs_layout_passes=False
    ),
)
def _fused_kernel(
    idx_hbm,
    tt_hbm,
    tail_hbm,
    out_hbm,
    rt_hbm,
    buf,
    tbuf,
    tailbuf,
    idx_v,
    rows_v,
    x_p,
    dsem,
    xsem,
):
    cid = lax.axis_index("c")
    sid = lax.axis_index("s")
    wid = sid * _NC + cid

    lane = lax.broadcasted_iota(jnp.int32, (16,), 0)

    def _unit(i, carry):
        u = i * _NW + wid

        @pl.when(u < _FULL_UNITS)
        def _full():
            pltpu.sync_copy(tt_hbm.at[:, pl.ds(u * 128, 128)], buf)

            for d in range(EMBED_DIM):
                dvec = jnp.full((16,), d, jnp.int32)
                for cb in range(8):
                    v = buf[d, pl.ds(cb * 16, 16)]
                    plsc.store_scatter(tbuf, [cb * 16 + lane, dvec], v)
            pltpu.sync_copy(tbuf, rt_hbm.at[pl.ds(u * 128, 128)])

        return carry

    lax.fori_loop(0, _UPW, _unit, 0)

    @pl.when(wid == _NW - 1)
    def _tail():
        pltpu.sync_copy(tail_hbm, tailbuf)
        for t in range(_TAIL):
            for h in range(2):
                tbuf[t, pl.ds(h * 16, 16)] = tailbuf[t, pl.ds(h * 16, 16)]
        pltpu.sync_copy(
            tbuf.at[pl.ds(0, _TAIL)], rt_hbm.at[pl.ds(_FULL_UNITS * 128, _TAIL)]
        )

    plsc.subcore_barrier()

    @pl.when(sid == 0)
    def _cross_core():
        pl.semaphore_signal(xsem, 1, core_index=1 - cid)
        pl.semaphore_wait(xsem, 1)

    plsc.subcore_barrier()

    base = wid * _N
    pltpu.sync_copy(idx_hbm.at[pl.ds(base, _N)], idx_v)
    pltpu.async_copy(rt_hbm.at[idx_v], rows_v, dsem).wait()

    r_lo = lane // 8
    sr_lo = lane % 8
    r_hi = (lane + 16) // 8
    sr_hi = (lane + 16) % 8

    def _emit(j, carry):
        lo = rows_v[j, pl.ds(0, 16)]
        hi = rows_v[j, pl.ds(16, 16)]
        cc = jnp.full((16,), j // 128, jnp.int32)
        l = jnp.full((16,), j % 128, jnp.int32)
        plsc.store_scatter(x_p, [r_lo, cc, sr_lo, l], lo)
        plsc.store_scatter(x_p, [r_hi, cc, sr_hi, l], hi)
        return carry

    lax.fori_loop(0, _N, _emit, 0)
    pltpu.sync_copy(
        x_p.at[:, :, :, pl.ds(0, 128)], out_hbm.at[:, pl.ds(wid * _CW, _CW)]
    )


def kernel(inputs, table):
    tail = jnp.pad(table[_FULL_UNITS * 128 :], ((0, 40 - _TAIL), (0, 96)))
    x = _fused_kernel(inputs.astype(jnp.int32), table.T, tail)
    return jnp.transpose(x, (1, 3, 0, 2)).reshape(BATCH, EMBED_DIM)


# speedup vs baseline: 1.6201x; 1.6201x over previous
import functools

import jax
import jax.numpy as jnp
from jax import lax
from jax.experimental import pallas as pl
from jax.experimental.pallas import tpu as pltpu
from jax.experimental.pallas import tpu_sc as plsc

VOCAB = 100001
EMBED_DIM = 32
BATCH = 16384

_info = plsc.get_sparse_core_info()
_NC = _info.num_cores
_NS = _info.num_subcores
_NW = _NC * _NS
_N = BATCH // _NW

_R = EMBED_DIM // 8
_C = BATCH // 128
_CW = _C // _NW

_mesh = plsc.VectorSubcoreMesh(core_axis_name="c", subcore_axis_name="s")


@functools.partial(
    pl.kernel,
    mesh=_mesh,
    out_type=jax.ShapeDtypeStruct((_R, _C, 8, 128), jnp.float32),
    scratch_types=[
        pltpu.VMEM((_N,), jnp.int32),
        pltpu.VMEM((_N, EMBED_DIM), jnp.float32),
        pltpu.VMEM((_R, _CW, 8, 128), jnp.float32),
        pltpu.SemaphoreType.DMA,
    ],
    compiler_params=pltpu.CompilerParams(
        use_tc_tiling_on_sc=False, needs_layout_passes=False
    ),
)
def _gather_kernel(idx_hbm, table_hbm, out_hbm, idx_v, rows_v, x_l, sem):
    wid = lax.axis_index("s") * _NC + lax.axis_index("c")
    base = wid * _N
    pltpu.sync_copy(idx_hbm.at[pl.ds(base, _N)], idx_v)
    pltpu.async_copy(table_hbm.at[idx_v], rows_v, sem).wait()

    lane = lax.broadcasted_iota(jnp.int32, (16,), 0)

    def _rearrange(cc, carry):
        for r in range(_R):
            for sr in range(8):
                col = jnp.full((16,), 8 * r + sr, jnp.int32)
                for lb in range(8):
                    row_idx = cc * 128 + lb * 16 + lane
                    x_l[r, cc, sr, pl.ds(lb * 16, 16)] = plsc.load_gather(
                        rows_v, [row_idx, col]
                    )
        return carry

    lax.fori_loop(0, _CW, _rearrange, 0)
    pltpu.sync_copy(x_l, out_hbm.at[:, pl.ds(wid * _CW, _CW)])


def kernel(inputs, table):
    x = _gather_kernel(inputs.astype(jnp.int32), table)
    return jnp.transpose(x, (1, 3, 0, 2)).reshape(BATCH, EMBED_DIM)
